# Initial kernel scaffold; baseline (speedup 1.0000x reference)
import jax
import jax.numpy as jnp
from jax.experimental import pallas as pl


def kernel(token_ids, weight):
    raise NotImplementedError("write your pallas kernel here")


# speedup vs baseline: 1.6861x; 1.6861x over previous
import jax
import jax.numpy as jnp
from jax import lax
from jax.experimental import pallas as pl
from jax.experimental.pallas import tpu as pltpu
from jax.experimental.pallas import tpu_sc as plsc

_D = 64
_NC, _NS = 2, 16
_NW = _NC * _NS
_CHUNK = 128


def _emb_body(idx_hbm, table_hbm, out_hbm, idx_v, rows_v, gsem):
    wid = lax.axis_index("s") * _NC + lax.axis_index("c")
    n_chunk = idx_hbm.shape[1]
    base = wid * (n_chunk * _CHUNK)
    pltpu.sync_copy(idx_hbm.at[wid], idx_v)

    def body(j, carry):
        pltpu.async_copy(table_hbm.at[idx_v.at[j]], rows_v, gsem).wait()
        pltpu.sync_copy(rows_v, out_hbm.at[pl.ds(base + j * _CHUNK, _CHUNK)])
        return carry

    lax.fori_loop(0, n_chunk, body, 0)


def kernel(token_ids, weight):
    b, s = token_ids.shape
    total = b * s
    n_chunk = total // (_NW * _CHUNK)
    idx = token_ids.reshape(_NW, n_chunk, _CHUNK).astype(jnp.int32)
    mesh = plsc.VectorSubcoreMesh(core_axis_name="c", subcore_axis_name="s")
    out = pl.kernel(
        _emb_body,
        out_type=jax.ShapeDtypeStruct((total, _D), jnp.float32),
        mesh=mesh,
        scratch_types=[
            pltpu.VMEM((n_chunk, _CHUNK), jnp.int32),
            pltpu.VMEM((_CHUNK, _D), jnp.float32),
            pltpu.SemaphoreType.DMA,
        ],
        compiler_params=pltpu.CompilerParams(use_tc_tiling_on_sc=False),
    )(idx, weight)
    return out.reshape(b, s, _D)


# speedup vs baseline: 1.8777x; 1.1136x over previous
import jax
import jax.numpy as jnp
from jax import lax
from jax.experimental import pallas as pl
from jax.experimental.pallas import tpu as pltpu
from jax.experimental.pallas import tpu_sc as plsc

_D = 64
_NC, _NS = 2, 16
_NW = _NC * _NS
_CHUNK = 128
_NBUF = 8


def _emb_body(idx_hbm, table_hbm, out_hbm, idx_v, rows_v, gsem, osem):
    wid = lax.axis_index("s") * _NC + lax.axis_index("c")
    n_chunk = idx_hbm.shape[1]
    base = wid * (n_chunk * _CHUNK)
    pltpu.sync_copy(idx_hbm.at[wid], idx_v)

    def start_gather(j, b):
        pltpu.async_copy(table_hbm.at[idx_v.at[j]], rows_v.at[b], gsem.at[b])

    def wait_gather(j, b):
        pltpu.make_async_copy(
            table_hbm.at[idx_v.at[j]], rows_v.at[b], gsem.at[b]).wait()

    def out_slice(j):
        return out_hbm.at[pl.ds(base + j * _CHUNK, _CHUNK)]

    def start_out(j, b):
        pltpu.async_copy(rows_v.at[b], out_slice(j), osem.at[b])

    def wait_out(j, b):
        pltpu.make_async_copy(rows_v.at[b], out_slice(j), osem.at[b]).wait()

    for b in range(_NBUF):
        start_gather(b, b)

    n_group = n_chunk // _NBUF

    def body(g, carry):
        for b in range(_NBUF):
            j = g * _NBUF + b
            wait_gather(j, b)
            start_out(j, b)
            wait_out(j, b)
            start_gather(j + _NBUF, b)
        return carry

    lax.fori_loop(0, n_group - 1, body, 0)

    for b in range(_NBUF):
        j = (n_group - 1) * _NBUF + b
        wait_gather(j, b)
        start_out(j, b)
    for b in range(_NBUF):
        j = (n_group - 1) * _NBUF + b
        wait_out(j, b)


def kernel(token_ids, weight):
    b, s = token_ids.shape
    total = b * s
    n_chunk = total // (_NW * _CHUNK)
    idx = token_ids.reshape(_NW, n_chunk, _CHUNK).astype(jnp.int32)
    mesh = plsc.VectorSubcoreMesh(core_axis_name="c", subcore_axis_name="s")
    out = pl.kernel(
        _emb_body,
        out_type=jax.ShapeDtypeStruct((total, _D), jnp.float32),
        mesh=mesh,
        scratch_types=[
            pltpu.VMEM((n_chunk, _CHUNK), jnp.int32),
            pltpu.VMEM((_NBUF, _CHUNK, _D), jnp.float32),
            pltpu.SemaphoreType.DMA((_NBUF,)),
            pltpu.SemaphoreType.DMA((_NBUF,)),
        ],
        compiler_params=pltpu.CompilerParams(use_tc_tiling_on_sc=False),
    )(idx, weight)
    return out.reshape(b, s, _D)
